# baseline (speedup 1.0000x reference)
import functools

import jax
import jax.numpy as jnp
from jax import lax
from jax.experimental import pallas as pl
from jax.experimental.pallas import tpu as pltpu
from jax.experimental.pallas import tpu_sc as plsc

D = 64
T = 200
B = 1024
V = 100000
NC = 2
NS = 16
NW = NC * NS

TSLAB = 8
NSLAB = T // TSLAB
NPAIR = (NSLAB - 1) // 2


@functools.lru_cache(maxsize=None)
def _make_kernel():
    mesh = plsc.VectorSubcoreMesh(core_axis_name="c", subcore_axis_name="s")

    @functools.partial(
        pl.kernel,
        mesh=mesh,
        compiler_params=pltpu.CompilerParams(
            use_tc_tiling_on_sc=False, needs_layout_passes=False),
        out_type=jax.ShapeDtypeStruct((T, 8, 8, 8, 128), jnp.float32),
        scratch_types=[
            pltpu.VMEM((V,), jnp.float32),
            pltpu.VMEM((2, TSLAB, 8, 128), jnp.float32),
            pltpu.VMEM((T + 16,), jnp.float32),
            pltpu.VMEM_SHARED((T, 8, 128), jnp.float32),
            pltpu.SemaphoreType.DMA,
            pltpu.SemaphoreType.DMA,
            pltpu.SemaphoreType.DMA,
            pltpu.SemaphoreType.DMA,
        ],
    )
    def k(tab_hbm, xt_hbm, pos_hbm, out_hbm, row_v, buf_v, pos_v, x_sp,
          sx0, sx1, so0, so1):
        cid = lax.axis_index("c")
        sid = lax.axis_index("s")
        wid = sid * NC + cid
        semx = (sx0, sx1)
        semo = (so0, so1)

        @pl.when(sid == 0)
        def _():
            pltpu.sync_copy(xt_hbm, x_sp)
        plsc.subcore_barrier()

        def start_x(s8, r):
            t0 = pl.multiple_of(s8 * TSLAB, TSLAB)
            pltpu.async_copy(x_sp.at[pl.ds(t0, TSLAB)], buf_v.at[r], semx[r])

        def wait_x(r):
            pltpu.make_async_copy(x_sp.at[pl.ds(0, TSLAB)], buf_v.at[r],
                                  semx[r]).wait()

        def start_out(s8, r, j):
            t0 = pl.multiple_of(s8 * TSLAB, TSLAB)
            pltpu.async_copy(
                buf_v.at[r],
                out_hbm.at[pl.ds(t0, TSLAB), j // 8, :, j % 8],
                semo[r])

        def wait_out(r, j):
            pltpu.make_async_copy(
                buf_v.at[r],
                out_hbm.at[pl.ds(0, TSLAB), j // 8, :, j % 8],
                semo[r]).wait()

        def compute(s8, r, j):
            t0 = pl.multiple_of(s8 * TSLAB, TSLAB)
            pv16 = pos_v[pl.ds(t0, 16)]
            for tt in range(TSLAB):
                pos_s = pv16[tt]

                @plsc.parallel_loop(0, B // 16, unroll=16)
                def _(vv):
                    cc = vv // 8
                    c0 = (vv % 8) * 16
                    idx16 = plsc.bitcast(buf_v[r, tt, cc, pl.ds(c0, 16)],
                                         jnp.int32)
                    g = plsc.load_gather(row_v, [idx16])
                    buf_v[r, tt, cc, pl.ds(c0, 16)] = g + pos_s

        for p in range(D // NW):
            j = p * NW + wid
            pltpu.sync_copy(tab_hbm.at[j], row_v)
            pltpu.sync_copy(pos_hbm.at[j], pos_v.at[pl.ds(0, T)])
            start_x(0, 0)

            def pair_body(g, carry):
                s8 = g * 2
                wait_x(0)

                @pl.when(g > 0)
                def _():
                    wait_out(1, j)
                start_x(s8 + 1, 1)
                compute(s8, 0, j)
                start_out(s8, 0, j)
                wait_x(1)
                wait_out(0, j)
                start_x(s8 + 2, 0)
                compute(s8 + 1, 1, j)
                start_out(s8 + 1, 1, j)
                return carry

            lax.fori_loop(0, NPAIR, pair_body, 0)
            wait_x(0)
            wait_out(1, j)
            compute(NSLAB - 1, 0, j)
            start_out(NSLAB - 1, 0, j)
            wait_out(0, j)

    return k


def kernel(x, token_table, pos_table):
    xt = jax.lax.bitcast_convert_type(
        x.T.astype(jnp.int32).reshape(T, 8, 128), jnp.float32)
    out5 = _make_kernel()(token_table.T, xt, pos_table.T)
    out_t = jnp.transpose(out5, (0, 1, 3, 2, 4)).reshape(T, D, B)
    return jnp.transpose(out_t, (2, 0, 1))


# speedup vs baseline: 1.0125x; 1.0125x over previous
import functools

import jax
import jax.numpy as jnp
from jax import lax
from jax.experimental import pallas as pl
from jax.experimental.pallas import tpu as pltpu
from jax.experimental.pallas import tpu_sc as plsc

D = 64
T = 200
B = 1024
V = 100000
NC = 2
NS = 16
NW = NC * NS

TSLAB = 8
NSLAB = T // TSLAB
NPAIR = (NSLAB - 1) // 2


@functools.lru_cache(maxsize=None)
def _make_kernel():
    mesh = plsc.VectorSubcoreMesh(core_axis_name="c", subcore_axis_name="s")

    @functools.partial(
        pl.kernel,
        mesh=mesh,
        compiler_params=pltpu.CompilerParams(
            use_tc_tiling_on_sc=False, needs_layout_passes=False),
        out_type=jax.ShapeDtypeStruct((T, 8, 8, 8, 128), jnp.float32),
        scratch_types=[
            pltpu.VMEM((V,), jnp.float32),
            pltpu.VMEM((2, TSLAB, 8, 128), jnp.float32),
            pltpu.VMEM((T + 16,), jnp.float32),
            pltpu.VMEM_SHARED((T, 8, 128), jnp.float32),
            pltpu.SemaphoreType.DMA,
            pltpu.SemaphoreType.DMA,
            pltpu.SemaphoreType.DMA,
            pltpu.SemaphoreType.DMA,
        ],
    )
    def k(tab_hbm, xt_hbm, pos_hbm, out_hbm, row_v, buf_v, pos_v, x_sp,
          sx0, sx1, so0, so1):
        cid = lax.axis_index("c")
        sid = lax.axis_index("s")
        wid = sid * NC + cid
        semx = (sx0, sx1)
        semo = (so0, so1)

        @pl.when(sid == 0)
        def _():
            pltpu.sync_copy(xt_hbm, x_sp)
        plsc.subcore_barrier()

        def start_x(s8, r):
            t0 = pl.multiple_of(s8 * TSLAB, TSLAB)
            pltpu.async_copy(x_sp.at[pl.ds(t0, TSLAB)], buf_v.at[r], semx[r])

        def wait_x(r):
            pltpu.make_async_copy(x_sp.at[pl.ds(0, TSLAB)], buf_v.at[r],
                                  semx[r]).wait()

        def start_out(s8, r, j):
            t0 = pl.multiple_of(s8 * TSLAB, TSLAB)
            pltpu.async_copy(
                buf_v.at[r],
                out_hbm.at[pl.ds(t0, TSLAB), j // 8, :, j % 8],
                semo[r])

        def wait_out(r, j):
            pltpu.make_async_copy(
                buf_v.at[r],
                out_hbm.at[pl.ds(0, TSLAB), j // 8, :, j % 8],
                semo[r]).wait()

        def compute(s8, r, j):
            t0 = pl.multiple_of(s8 * TSLAB, TSLAB)
            pv16 = pos_v[pl.ds(t0, 16)]
            for tt in range(TSLAB):
                pos_s = pv16[tt]

                @plsc.parallel_loop(0, B // 16, unroll=8)
                def _(vv):
                    cc = vv // 8
                    c0 = (vv % 8) * 16
                    idx16 = plsc.bitcast(buf_v[r, tt, cc, pl.ds(c0, 16)],
                                         jnp.int32)
                    g = plsc.load_gather(row_v, [idx16])
                    buf_v[r, tt, cc, pl.ds(c0, 16)] = g + pos_s

        for p in range(D // NW):
            j = p * NW + wid
            pltpu.sync_copy(tab_hbm.at[j], row_v)
            pltpu.sync_copy(pos_hbm.at[j], pos_v.at[pl.ds(0, T)])
            start_x(0, 0)

            def pair_body(g, carry):
                s8 = g * 2
                wait_x(0)

                @pl.when(g > 0)
                def _():
                    wait_out(1, j)
                start_x(s8 + 1, 1)
                compute(s8, 0, j)
                start_out(s8, 0, j)
                wait_x(1)
                wait_out(0, j)
                start_x(s8 + 2, 0)
                compute(s8 + 1, 1, j)
                start_out(s8 + 1, 1, j)
                return carry

            lax.fori_loop(0, NPAIR, pair_body, 0)
            wait_x(0)
            wait_out(1, j)
            compute(NSLAB - 1, 0, j)
            start_out(NSLAB - 1, 0, j)
            wait_out(0, j)

    return k


def kernel(x, token_table, pos_table):
    xt = jax.lax.bitcast_convert_type(
        x.T.astype(jnp.int32).reshape(T, 8, 128), jnp.float32)
    out5 = _make_kernel()(token_table.T, xt, pos_table.T)
    out_t = jnp.transpose(out5, (0, 1, 3, 2, 4)).reshape(T, D, B)
    return jnp.transpose(out_t, (2, 0, 1))
